# baseline (speedup 1.0000x reference)
import functools

import jax
import jax.numpy as jnp
from jax import lax
from jax.experimental import pallas as pl
from jax.experimental.pallas import tpu as pltpu
from jax.experimental.pallas import tpu_sc as plsc

VS_ = 1000000
DS_ = 64
B_ = 16384
C_ = 20

NC = 2
NS = 16
NW = NC * NS
CHUNK = 128
NBUF = 2

B_W = B_ // NW
WRD_CH_W = B_W // CHUNK
CTX_CH_W = (B_ * C_ // CHUNK) // NW
ERR_W = B_W * C_

CW = 4096
KH = 124 * CW


def _tc_repack_body(a_ref, b_ref, out_ref):
    out_ref[:, pl.ds(0, DS_)] = a_ref[...].T
    out_ref[:, pl.ds(DS_, DS_)] = b_ref[...].T


@functools.lru_cache(maxsize=1)
def _build_tc_repack():
    nb = KH // CW
    return pl.pallas_call(
        _tc_repack_body,
        grid=(nb,),
        in_specs=[
            pl.BlockSpec((DS_, CW), lambda i: (0, i)),
            pl.BlockSpec((DS_, CW),
                         lambda i: (0, jnp.minimum(i + KH // CW, VS_ // CW))),
        ],
        out_specs=pl.BlockSpec((CW, 2 * DS_), lambda i: (i, 0)),
        out_shape=jax.ShapeDtypeStruct((KH, 2 * DS_), jnp.float32),
    )


def _gather_dot_body(wrd_idx, ctx_idx, pi, po, out_e, widx, cidx, qbuf,
                     wrows, rows, estage, *sems):
    gsems = sems[:NBUF]
    wid = lax.axis_index("s") * NC + lax.axis_index("c")
    lane = lax.iota(jnp.int32, 16)
    last = jnp.full((16,), 15, jnp.int32)

    pltpu.sync_copy(wrd_idx.at[pl.ds(wid * WRD_CH_W, WRD_CH_W)], widx)
    pltpu.sync_copy(ctx_idx.at[pl.ds(wid * CTX_CH_W, CTX_CH_W)], cidx)

    def shift_row(src, j, b):
        for g in range(CHUNK // 16):
            v = src[j, pl.ds(g * 16, 16)]
            qbuf[b, pl.ds(g * 16, 16)] = jnp.where(v >= KH, v - KH, v)

    for c in range(WRD_CH_W):
        shift_row(widx, c, 0)
        pltpu.async_copy(pi.at[qbuf.at[0]], rows.at[0], gsems[0])
        pltpu.make_async_copy(pi.at[qbuf.at[0]], rows.at[0], gsems[0]).wait()

        def wcomp(g, carry):
            r0 = g * 16
            pv = widx[c, pl.ds(r0, 16)]
            for t in range(16):
                off = jnp.where(pv[t] >= KH, DS_, 0)
                r = r0 + t
                for k in range(4):
                    wrows[c * CHUNK + r, pl.ds(k * 16, 16)] = (
                        rows[0, r, pl.ds(off + k * 16, 16)])
            return carry
        lax.fori_loop(0, CHUNK // 16, wcomp, 0)

    def start_gather(j, b):
        shift_row(cidx, j, b)
        pltpu.async_copy(po.at[qbuf.at[b]], rows.at[b], gsems[b])

    def wait_gather(b):
        pltpu.make_async_copy(po.at[qbuf.at[0]], rows.at[b], gsems[b]).wait()

    def consume(j, b):
        def grp(g, carry):
            r0 = g * 16
            pv = cidx[j, pl.ds(r0, 16)]
            vec = jnp.zeros((16,), jnp.float32)
            for t in range(16):
                r = r0 + t
                off = jnp.where(pv[t] >= KH, DS_, 0)
                bl = (j * CHUNK + r) // C_
                acc = (wrows[bl, pl.ds(0, 16)]
                       * rows[b, r, pl.ds(off, 16)]
                       + wrows[bl, pl.ds(16, 16)]
                       * rows[b, r, pl.ds(off + 16, 16)]
                       + wrows[bl, pl.ds(32, 16)]
                       * rows[b, r, pl.ds(off + 32, 16)]
                       + wrows[bl, pl.ds(48, 16)]
                       * rows[b, r, pl.ds(off + 48, 16)])
                cs = plsc.cumsum(acc)
                sv = cs.at[last].get(mode="promise_in_bounds")
                vec = jnp.where(lane == t, sv, vec)
            estage[pl.ds(j * CHUNK + r0, 16)] = vec
            return carry
        lax.fori_loop(0, CHUNK // 16, grp, 0)

    for b in range(NBUF):
        start_gather(b, b)

    n_groups = CTX_CH_W // NBUF

    def group(gi, carry):
        for b in range(NBUF):
            j = gi * NBUF + b
            wait_gather(b)
            consume(j, b)
            start_gather(j + NBUF, b)
        return carry
    lax.fori_loop(0, n_groups - 1, group, 0)

    for b in range(NBUF):
        j = (n_groups - 1) * NBUF + b
        wait_gather(b)
        consume(j, b)

    pltpu.sync_copy(estage, out_e.at[pl.ds(wid * ERR_W, ERR_W)])


RB = 2048


def _tc_loss_body(err_ref, pos_ref, neg_ref, out_ref):
    e = err_ref[...]
    p = pos_ref[...]
    n = neg_ref[...]
    e = e * (1.0 - 2.0 * n)
    sg = 1.0 / (1.0 + jnp.exp(-e))
    l = -jnp.log(jnp.clip(sg, 1e-6, 1.0 - 1e-6))
    pe = jnp.sum(l * p, axis=1) / jnp.sum(p, axis=1)
    ne = jnp.sum(l * n, axis=1)
    blk = jnp.sum(pe) + jnp.sum(ne)

    @pl.when(pl.program_id(0) == 0)
    def _():
        out_ref[0, 0] = 0.0

    out_ref[0, 0] += blk


@functools.lru_cache(maxsize=1)
def _build_gather_dot():
    mesh = plsc.VectorSubcoreMesh(core_axis_name="c", subcore_axis_name="s")
    return pl.kernel(
        _gather_dot_body,
        out_type=jax.ShapeDtypeStruct((B_ * C_,), jnp.float32),
        mesh=mesh,
        scratch_types=[
            pltpu.VMEM((WRD_CH_W, CHUNK), jnp.int32),
            pltpu.VMEM((CTX_CH_W, CHUNK), jnp.int32),
            pltpu.VMEM((NBUF, CHUNK), jnp.int32),
            pltpu.VMEM((B_W, DS_), jnp.float32),
            pltpu.VMEM((NBUF, CHUNK, 2 * DS_), jnp.float32),
            pltpu.VMEM((ERR_W,), jnp.float32),
        ] + [pltpu.SemaphoreType.DMA] * NBUF,
        compiler_params=pltpu.CompilerParams(use_tc_tiling_on_sc=True,
                                             needs_layout_passes=False),
    )


@functools.lru_cache(maxsize=1)
def _build_tc_loss():
    grid = B_ // RB
    return pl.pallas_call(
        _tc_loss_body,
        grid=(grid,),
        in_specs=[
            pl.BlockSpec((RB, C_), lambda i: (i, 0)),
            pl.BlockSpec((RB, C_), lambda i: (i, 0)),
            pl.BlockSpec((RB, C_), lambda i: (i, 0)),
        ],
        out_specs=pl.BlockSpec((1, 1), lambda i: (0, 0),
                               memory_space=pltpu.SMEM),
        out_shape=jax.ShapeDtypeStruct((1, 1), jnp.float32),
    )


def kernel(wrd, ctx, pos, neg, W_i, W_o):
    wrd2d = wrd.astype(jnp.int32).reshape(B_ // CHUNK, CHUNK)
    ctx2d = ctx.astype(jnp.int32).reshape(B_ * C_ // CHUNK, CHUNK)
    wit = W_i.T
    wot = W_o.T
    p_i = _build_tc_repack()(wit, wit)
    p_o = _build_tc_repack()(wot, wot)
    err = _build_gather_dot()(wrd2d, ctx2d, p_i, p_o)
    tot = _build_tc_loss()(err.reshape(B_, C_), pos, neg)
    return tot[0, 0] / B_


# speedup vs baseline: 1.1104x; 1.1104x over previous
import functools

import jax
import jax.numpy as jnp
from jax import lax
from jax.experimental import pallas as pl
from jax.experimental.pallas import tpu as pltpu
from jax.experimental.pallas import tpu_sc as plsc

VS_ = 1000000
DS_ = 64
B_ = 16384
C_ = 20

NC = 2
NS = 16
NW = NC * NS
CHUNK = 128
NBUF = 2

B_W = B_ // NW
WRD_CH_W = B_W // CHUNK
CTX_CH_W = (B_ * C_ // CHUNK) // NW
ERR_W = B_W * C_

CW = 8192
KH = 62 * CW


def _tc_repack_body(a_ref, b_ref, out_ref):
    out_ref[:, pl.ds(0, DS_)] = a_ref[...].T
    out_ref[:, pl.ds(DS_, DS_)] = b_ref[...].T


@functools.lru_cache(maxsize=1)
def _build_tc_repack():
    nb = KH // CW
    return pl.pallas_call(
        _tc_repack_body,
        grid=(nb,),
        in_specs=[
            pl.BlockSpec((DS_, CW), lambda i: (0, i)),
            pl.BlockSpec((DS_, CW),
                         lambda i: (0, jnp.minimum(i + KH // CW, VS_ // CW))),
        ],
        out_specs=pl.BlockSpec((CW, 2 * DS_), lambda i: (i, 0)),
        out_shape=jax.ShapeDtypeStruct((KH, 2 * DS_), jnp.float32),
    )


def _gather_dot_body(wrd_idx, ctx_idx, pi, po, out_e, widx, cidx, qbuf,
                     wrows, rows, estage, *sems):
    gsems = sems[:NBUF]
    wid = lax.axis_index("s") * NC + lax.axis_index("c")
    lane = lax.iota(jnp.int32, 16)
    last = jnp.full((16,), 15, jnp.int32)

    pltpu.sync_copy(wrd_idx.at[pl.ds(wid * WRD_CH_W, WRD_CH_W)], widx)
    pltpu.sync_copy(ctx_idx.at[pl.ds(wid * CTX_CH_W, CTX_CH_W)], cidx)

    def shift_row(src, j, b):
        for g in range(CHUNK // 16):
            v = src[j, pl.ds(g * 16, 16)]
            qbuf[b, pl.ds(g * 16, 16)] = jnp.where(v >= KH, v - KH, v)

    for c in range(WRD_CH_W):
        shift_row(widx, c, 0)
        pltpu.async_copy(pi.at[qbuf.at[0]], rows.at[0], gsems[0])
        pltpu.make_async_copy(pi.at[qbuf.at[0]], rows.at[0], gsems[0]).wait()

        def wcomp(g, carry):
            r0 = g * 16
            pv = widx[c, pl.ds(r0, 16)]
            for t in range(16):
                off = jnp.where(pv[t] >= KH, DS_, 0)
                r = r0 + t
                for k in range(4):
                    wrows[c * CHUNK + r, pl.ds(k * 16, 16)] = (
                        rows[0, r, pl.ds(off + k * 16, 16)])
            return carry
        lax.fori_loop(0, CHUNK // 16, wcomp, 0)

    def start_gather(j, b):
        shift_row(cidx, j, b)
        pltpu.async_copy(po.at[qbuf.at[b]], rows.at[b], gsems[b])

    def wait_gather(b):
        pltpu.make_async_copy(po.at[qbuf.at[0]], rows.at[b], gsems[b]).wait()

    def consume(j, b):
        def grp(g, carry):
            r0 = g * 16
            pv = cidx[j, pl.ds(r0, 16)]
            vec = jnp.zeros((16,), jnp.float32)
            for t in range(16):
                r = r0 + t
                off = jnp.where(pv[t] >= KH, DS_, 0)
                bl = (j * CHUNK + r) // C_
                acc = (wrows[bl, pl.ds(0, 16)]
                       * rows[b, r, pl.ds(off, 16)]
                       + wrows[bl, pl.ds(16, 16)]
                       * rows[b, r, pl.ds(off + 16, 16)]
                       + wrows[bl, pl.ds(32, 16)]
                       * rows[b, r, pl.ds(off + 32, 16)]
                       + wrows[bl, pl.ds(48, 16)]
                       * rows[b, r, pl.ds(off + 48, 16)])
                cs = plsc.cumsum(acc)
                sv = cs.at[last].get(mode="promise_in_bounds")
                vec = jnp.where(lane == t, sv, vec)
            estage[pl.ds(j * CHUNK + r0, 16)] = vec
            return carry
        lax.fori_loop(0, CHUNK // 16, grp, 0)

    for b in range(NBUF):
        start_gather(b, b)

    n_groups = CTX_CH_W // NBUF

    def group(gi, carry):
        for b in range(NBUF):
            j = gi * NBUF + b
            wait_gather(b)
            consume(j, b)
            start_gather(j + NBUF, b)
        return carry
    lax.fori_loop(0, n_groups - 1, group, 0)

    for b in range(NBUF):
        j = (n_groups - 1) * NBUF + b
        wait_gather(b)
        consume(j, b)

    pltpu.sync_copy(estage, out_e.at[pl.ds(wid * ERR_W, ERR_W)])


RB = 2048


def _tc_loss_body(err_ref, pos_ref, neg_ref, out_ref):
    e = err_ref[...]
    p = pos_ref[...]
    n = neg_ref[...]
    e = e * (1.0 - 2.0 * n)
    sg = 1.0 / (1.0 + jnp.exp(-e))
    l = -jnp.log(jnp.clip(sg, 1e-6, 1.0 - 1e-6))
    pe = jnp.sum(l * p, axis=1) / jnp.sum(p, axis=1)
    ne = jnp.sum(l * n, axis=1)
    blk = jnp.sum(pe) + jnp.sum(ne)

    @pl.when(pl.program_id(0) == 0)
    def _():
        out_ref[0, 0] = 0.0

    out_ref[0, 0] += blk


@functools.lru_cache(maxsize=1)
def _build_gather_dot():
    mesh = plsc.VectorSubcoreMesh(core_axis_name="c", subcore_axis_name="s")
    return pl.kernel(
        _gather_dot_body,
        out_type=jax.ShapeDtypeStruct((B_ * C_,), jnp.float32),
        mesh=mesh,
        scratch_types=[
            pltpu.VMEM((WRD_CH_W, CHUNK), jnp.int32),
            pltpu.VMEM((CTX_CH_W, CHUNK), jnp.int32),
            pltpu.VMEM((NBUF, CHUNK), jnp.int32),
            pltpu.VMEM((B_W, DS_), jnp.float32),
            pltpu.VMEM((NBUF, CHUNK, 2 * DS_), jnp.float32),
            pltpu.VMEM((ERR_W,), jnp.float32),
        ] + [pltpu.SemaphoreType.DMA] * NBUF,
        compiler_params=pltpu.CompilerParams(use_tc_tiling_on_sc=True,
                                             needs_layout_passes=False),
    )


@functools.lru_cache(maxsize=1)
def _build_tc_loss():
    grid = B_ // RB
    return pl.pallas_call(
        _tc_loss_body,
        grid=(grid,),
        in_specs=[
            pl.BlockSpec((RB, C_), lambda i: (i, 0)),
            pl.BlockSpec((RB, C_), lambda i: (i, 0)),
            pl.BlockSpec((RB, C_), lambda i: (i, 0)),
        ],
        out_specs=pl.BlockSpec((1, 1), lambda i: (0, 0),
                               memory_space=pltpu.SMEM),
        out_shape=jax.ShapeDtypeStruct((1, 1), jnp.float32),
    )


def kernel(wrd, ctx, pos, neg, W_i, W_o):
    wrd2d = wrd.astype(jnp.int32).reshape(B_ // CHUNK, CHUNK)
    ctx2d = ctx.astype(jnp.int32).reshape(B_ * C_ // CHUNK, CHUNK)
    wit = W_i.T
    wot = W_o.T
    p_i = _build_tc_repack()(wit, wit)
    p_o = _build_tc_repack()(wot, wot)
    err = _build_gather_dot()(wrd2d, ctx2d, p_i, p_o)
    tot = _build_tc_loss()(err.reshape(B_, C_), pos, neg)
    return tot[0, 0] / B_
